# Initial kernel scaffold; baseline (speedup 1.0000x reference)
import jax
import jax.numpy as jnp
from jax.experimental import pallas as pl


def kernel(x, edge_index, batch, W1, b1, W2, b2, W3, b3, Wlin, blin):
    raise NotImplementedError("write your pallas kernel here")


# speedup vs baseline: 9.0699x; 9.0699x over previous
import jax
import jax.numpy as jnp
from jax import lax
from jax.experimental import pallas as pl
from jax.experimental.pallas import tpu as pltpu
from jax.experimental.pallas import tpu_sc as plsc

N = 10000
E = 320000
D = 128
H = 128
G = 64
C = 2

NC = 2
NS = 16
NW = NC * NS
EDEG = E // NW
CH = 128
NCHW = 20
NWIN = 4
EPAD = NWIN * NCHW * CH
NA = N + 8

_mesh = plsc.VectorSubcoreMesh(
    core_axis_name="c", subcore_axis_name="s", num_cores=NC, num_subcores=NS)


def _deg_body(dst_hbm, zeros_hbm, out_hbm, dst_v, deg_v):
    c = lax.axis_index("c")
    s = lax.axis_index("s")
    wid = s * NC + c
    pltpu.sync_copy(dst_hbm.at[wid], dst_v)
    pltpu.sync_copy(zeros_hbm, deg_v)
    ones = jnp.ones((16,), jnp.float32)

    def body(k, carry):
        idx = dst_v[pl.ds(k * 16, 16)]
        plsc.addupdate_scatter(deg_v, [idx], ones)
        return carry

    lax.fori_loop(0, EDEG // 16, body, 0)
    pltpu.sync_copy(deg_v, out_hbm.at[wid])


_deg = pl.kernel(
    _deg_body,
    out_type=jax.ShapeDtypeStruct((NW, N), jnp.float32),
    mesh=_mesh,
    compiler_params=pltpu.CompilerParams(needs_layout_passes=False),
    scratch_types=[
        pltpu.VMEM((EDEG,), jnp.int32),
        pltpu.VMEM((N,), jnp.float32),
    ],
)


def _agg_body(y_hbm, src_hbm, dst_hbm, zeros_hbm, out_hbm,
              srcw0, srcw1, dstw0, dstw1, rows0, rows1, s_sh,
              sem_s0, sem_s1, sem_d0, sem_d1, sem_r0, sem_r1):
    c = lax.axis_index("c")
    s = lax.axis_index("s")
    srcw = (srcw0, srcw1)
    dstw = (dstw0, dstw1)
    sems = (sem_s0, sem_s1)
    semd = (sem_d0, sem_d1)

    @pl.when(s == 0)
    def _():
        pltpu.sync_copy(zeros_hbm, s_sh)

    pltpu.async_copy(src_hbm.at[c, s, 0], srcw0, sem_s0)
    pltpu.async_copy(dst_hbm.at[c, s, 0], dstw0, sem_d0)
    plsc.subcore_barrier()

    for w in range(NWIN):
        sw, dw = srcw[w % 2], dstw[w % 2]
        pltpu.make_async_copy(src_hbm.at[c, s, w], sw, sems[w % 2]).wait()
        pltpu.make_async_copy(dst_hbm.at[c, s, w], dw, semd[w % 2]).wait()
        if w + 1 < NWIN:
            pltpu.async_copy(
                src_hbm.at[c, s, w + 1], srcw[(w + 1) % 2], sems[(w + 1) % 2])
            pltpu.async_copy(
                dst_hbm.at[c, s, w + 1], dstw[(w + 1) % 2], semd[(w + 1) % 2])
        pltpu.async_copy(y_hbm.at[sw.at[0]], rows0, sem_r0)

        def pair(i, carry, sw=sw, dw=dw):
            j0 = 2 * i
            pltpu.make_async_copy(y_hbm.at[sw.at[j0]], rows0, sem_r0).wait()
            pltpu.async_copy(y_hbm.at[sw.at[j0 + 1]], rows1, sem_r1)
            pltpu.sync_copy(rows0, s_sh.at[dw.at[j0]], add=True)
            pltpu.make_async_copy(
                y_hbm.at[sw.at[j0 + 1]], rows1, sem_r1).wait()
            pltpu.async_copy(y_hbm.at[sw.at[j0 + 2]], rows0, sem_r0)
            pltpu.sync_copy(rows1, s_sh.at[dw.at[j0 + 1]], add=True)
            return carry

        lax.fori_loop(0, NCHW // 2 - 1, pair, 0)
        j0 = NCHW - 2
        pltpu.make_async_copy(y_hbm.at[sw.at[j0]], rows0, sem_r0).wait()
        pltpu.async_copy(y_hbm.at[sw.at[j0 + 1]], rows1, sem_r1)
        pltpu.sync_copy(rows0, s_sh.at[dw.at[j0]], add=True)
        pltpu.make_async_copy(y_hbm.at[sw.at[j0 + 1]], rows1, sem_r1).wait()
        pltpu.sync_copy(rows1, s_sh.at[dw.at[j0 + 1]], add=True)

    plsc.subcore_barrier()

    @pl.when(s == 0)
    def _():
        pltpu.sync_copy(s_sh.at[pl.ds(0, N)], out_hbm.at[c])


_agg = pl.kernel(
    _agg_body,
    out_type=jax.ShapeDtypeStruct((NC, N, H), jnp.float32),
    mesh=_mesh,
    scratch_types=[
        pltpu.VMEM((NCHW, CH), jnp.int32),
        pltpu.VMEM((NCHW, CH), jnp.int32),
        pltpu.VMEM((NCHW, CH), jnp.int32),
        pltpu.VMEM((NCHW, CH), jnp.int32),
        pltpu.VMEM((CH, H), jnp.float32),
        pltpu.VMEM((CH, H), jnp.float32),
        pltpu.VMEM_SHARED((NA, H), jnp.float32),
        pltpu.SemaphoreType.DMA,
        pltpu.SemaphoreType.DMA,
        pltpu.SemaphoreType.DMA,
        pltpu.SemaphoreType.DMA,
        pltpu.SemaphoreType.DMA,
        pltpu.SemaphoreType.DMA,
    ],
)


_R = 1000


def _tc1_body(degp_ref, x_ref, w_ref, dinv_ref, y_ref):
    deg = jnp.sum(degp_ref[...], axis=1, keepdims=True) + 1.0
    dinv = lax.rsqrt(deg)
    dinv_ref[...] = dinv
    xw = jnp.dot(x_ref[...], w_ref[...], preferred_element_type=jnp.float32)
    y_ref[...] = xw * dinv


def _tc1(degp_t, x, w):
    return pl.pallas_call(
        _tc1_body,
        grid=(N // _R,),
        in_specs=[
            pl.BlockSpec((_R, NW), lambda i: (i, 0)),
            pl.BlockSpec((_R, D), lambda i: (i, 0)),
            pl.BlockSpec((D, H), lambda i: (0, 0)),
        ],
        out_specs=[
            pl.BlockSpec((_R, 1), lambda i: (i, 0)),
            pl.BlockSpec((_R, H), lambda i: (i, 0)),
        ],
        out_shape=[
            jax.ShapeDtypeStruct((N, 1), jnp.float32),
            jax.ShapeDtypeStruct((N, H), jnp.float32),
        ],
    )(degp_t, x, w)


def _tcmid_body(s_ref, y_ref, dinv_ref, b_ref, w_ref, yn_ref):
    z = s_ref[0] + s_ref[1] + y_ref[...]
    dinv = dinv_ref[...]
    h = jnp.maximum(z * dinv + b_ref[...], 0.0)
    yn_ref[...] = jnp.dot(
        h, w_ref[...], preferred_element_type=jnp.float32) * dinv


def _tcmid(s, y, dinv, b, w):
    return pl.pallas_call(
        _tcmid_body,
        grid=(N // _R,),
        in_specs=[
            pl.BlockSpec((NC, _R, H), lambda i: (0, i, 0)),
            pl.BlockSpec((_R, H), lambda i: (i, 0)),
            pl.BlockSpec((_R, 1), lambda i: (i, 0)),
            pl.BlockSpec((1, H), lambda i: (0, 0)),
            pl.BlockSpec((H, H), lambda i: (0, 0)),
        ],
        out_specs=pl.BlockSpec((_R, H), lambda i: (i, 0)),
        out_shape=jax.ShapeDtypeStruct((N, H), jnp.float32),
    )(s, y, dinv, b, w)


def _tcf_body(s_ref, y_ref, dinv_ref, b_ref, batch_ref, wlin_ref, blin_ref,
              out_ref, pooled_acc, cnt_acc):
    i = pl.program_id(0)

    @pl.when(i == 0)
    def _():
        pooled_acc[...] = jnp.zeros_like(pooled_acc)
        cnt_acc[...] = jnp.zeros_like(cnt_acc)

    z = s_ref[0] + s_ref[1] + y_ref[...]
    h = z * dinv_ref[...] + b_ref[...]
    bb = batch_ref[0]
    gi = lax.broadcasted_iota(jnp.int32, (G, 1), 0)
    m = (bb == gi).astype(jnp.float32)
    pooled_acc[...] += jnp.dot(m, h, preferred_element_type=jnp.float32)
    cnt_acc[...] += jnp.sum(m, axis=1, keepdims=True)

    @pl.when(i == pl.num_programs(0) - 1)
    def _():
        pooled = pooled_acc[...] / jnp.maximum(cnt_acc[...], 1.0)
        out_ref[...] = jnp.dot(
            pooled, wlin_ref[...], preferred_element_type=jnp.float32
        ) + blin_ref[...]


def _tcf(s, y, dinv, b, batch3d, wlin, blin):
    return pl.pallas_call(
        _tcf_body,
        grid=(N // _R,),
        in_specs=[
            pl.BlockSpec((NC, _R, H), lambda i: (0, i, 0)),
            pl.BlockSpec((_R, H), lambda i: (i, 0)),
            pl.BlockSpec((_R, 1), lambda i: (i, 0)),
            pl.BlockSpec((1, H), lambda i: (0, 0)),
            pl.BlockSpec((1, 1, _R), lambda i: (i, 0, 0)),
            pl.BlockSpec((H, C), lambda i: (0, 0)),
            pl.BlockSpec((1, C), lambda i: (0, 0)),
        ],
        out_specs=pl.BlockSpec((G, C), lambda i: (0, 0)),
        out_shape=jax.ShapeDtypeStruct((G, C), jnp.float32),
        scratch_shapes=[
            pltpu.VMEM((G, H), jnp.float32),
            pltpu.VMEM((G, 1), jnp.float32),
        ],
    )(s, y, dinv, b, batch3d, wlin, blin)


def kernel(x, edge_index, batch, W1, b1, W2, b2, W3, b3, Wlin, blin):
    pad = EPAD - EDEG
    src_r = jnp.pad(edge_index[0].reshape(NW, EDEG), ((0, 0), (0, pad))
                    ).reshape(NC, NS, NWIN, NCHW, CH)
    dst_r = jnp.pad(edge_index[1].reshape(NW, EDEG), ((0, 0), (0, pad)),
                    constant_values=N).reshape(NC, NS, NWIN, NCHW, CH)
    dstd_r = edge_index[1].reshape(NW, EDEG)
    zeros2d = jnp.zeros((NA, H), jnp.float32)
    zeros1d = jnp.zeros((N,), jnp.float32)
    batch3d = batch.reshape(N // _R, 1, _R)

    degp = _deg(dstd_r, zeros1d)
    dinv, y = _tc1(degp.T, x, W1)
    s = _agg(y, src_r, dst_r, zeros2d)
    y = _tcmid(s, y, dinv, b1.reshape(1, H), W2)
    s = _agg(y, src_r, dst_r, zeros2d)
    y = _tcmid(s, y, dinv, b2.reshape(1, H), W3)
    s = _agg(y, src_r, dst_r, zeros2d)
    return _tcf(s, y, dinv, b3.reshape(1, H), batch3d,
                Wlin, blin.reshape(1, C))
